# Initial kernel scaffold; baseline (speedup 1.0000x reference)
import jax
import jax.numpy as jnp
from jax.experimental import pallas as pl


def kernel(x, mins, maxs, bin_counts, bin_edges, feature_ranges):
    raise NotImplementedError("write your pallas kernel here")


# speedup vs baseline: 1410.6087x; 1410.6087x over previous
import functools

import jax
import jax.numpy as jnp
from jax import lax
from jax.experimental import pallas as pl
from jax.experimental.pallas import tpu as pltpu
from jax.experimental.pallas import tpu_sc as plsc

_EPS = 1e-6
_NB = 8
_NC = _NB + 2
_L = 16


def _make_sc_hist(B, C, nworkers):
    rows_w = B // nworkers
    CHUNK = 16
    nchunks = rows_w // CHUNK
    nsteps = nchunks // 2
    ngroups = C // _L
    acc_len = _NC * C

    mesh = plsc.VectorSubcoreMesh(core_axis_name="c", subcore_axis_name="s")

    def body(x_hbm, sb_hbm, out_hbm, xb0, xb1, sb_v, acc_v, sem0, sem1):
        ncores = 2
        wid = lax.axis_index("s") * ncores + lax.axis_index("c")
        row0 = wid * rows_w

        def chunk_copy(i, buf, sem):
            off = (row0 + i * CHUNK) * C
            return pltpu.make_async_copy(
                x_hbm.at[pl.ds(off, CHUNK * C)], buf, sem)

        chunk_copy(0, xb0, sem0).start()
        chunk_copy(1, xb1, sem1).start()
        pltpu.sync_copy(sb_hbm, sb_v)

        zeros = jnp.zeros((_L,), jnp.float32)

        def zero_body(i, _):
            acc_v[pl.ds(i * _L, _L)] = zeros
            return _

        lax.fori_loop(0, acc_len // _L, zero_body, None)

        lane = lax.iota(jnp.int32, _L)
        ones = jnp.full((_L,), 1.0, jnp.float32)

        def compute(buf):
            def g_body(g, _):
                col0 = g * _L
                sv = sb_v[pl.ds(col0, _L)]
                bv = sb_v[pl.ds(C + col0, _L)]
                colv = col0 + lane
                for r in range(CHUNK):
                    v = buf[pl.ds(r * C + col0, _L)]
                    u = v * sv + bv
                    u = jnp.minimum(jnp.maximum(u, 0.0), 9.0)
                    idx = u.astype(jnp.int32)
                    fi = idx * C + colv
                    plsc.addupdate_scatter(acc_v, [fi], ones)
                return _

            lax.fori_loop(0, ngroups, g_body, None)

        def step(k, _):
            c0 = 2 * k
            chunk_copy(c0, xb0, sem0).wait()
            compute(xb0)

            @pl.when(k < nsteps - 1)
            def _start0():
                chunk_copy(c0 + 2, xb0, sem0).start()

            chunk_copy(c0 + 1, xb1, sem1).wait()
            compute(xb1)

            @pl.when(k < nsteps - 1)
            def _start1():
                chunk_copy(c0 + 3, xb1, sem1).start()

            return _

        lax.fori_loop(0, nsteps, step, None)

        pltpu.sync_copy(acc_v, out_hbm.at[wid])

    return pl.kernel(
        body,
        out_type=jax.ShapeDtypeStruct((nworkers, acc_len), jnp.float32),
        mesh=mesh,
        compiler_params=pltpu.CompilerParams(needs_layout_passes=False),
        scratch_types=[
            pltpu.VMEM((CHUNK * C,), jnp.float32),
            pltpu.VMEM((CHUNK * C,), jnp.float32),
            pltpu.VMEM((2 * C,), jnp.float32),
            pltpu.VMEM((acc_len,), jnp.float32),
            pltpu.SemaphoreType.DMA,
            pltpu.SemaphoreType.DMA,
        ],
    )


def kernel(x, mins, maxs, bin_counts, bin_edges, feature_ranges):
    B, C = x.shape
    info = plsc.get_sparse_core_info()
    nworkers = info.num_cores * info.num_subcores

    all_zero = jnp.all(bin_counts == 0.0)

    maxs2 = maxs + _EPS
    fr = maxs2 - mins
    mxs = maxs2 / fr
    mns = mins / fr
    span = mxs - mns
    scale_a = 8.0 / (fr * span)
    bias_a = 1.0 - 8.0 * mns / span

    inner = bin_edges[:, 1:-1]
    span_e = inner[:, -1] - inner[:, 0]
    safe = jnp.where(span_e != 0.0, span_e, 1.0)
    frq = feature_ranges[:, 0]
    scale_b = 8.0 / jnp.where(frq != 0.0, frq, 1.0) / safe
    bias_b = 1.0 - 8.0 * inner[:, 0] / safe

    scale = jnp.where(all_zero, scale_a, scale_b)
    bias = jnp.where(all_zero, bias_a, bias_b)
    sb = jnp.concatenate([scale, bias]).astype(jnp.float32)

    hist = _make_sc_hist(B, C, nworkers)
    parts = hist(x.reshape(-1), sb)
    counts = parts.sum(axis=0).reshape(_NC, C).T
    return (x, bin_counts + counts)


# speedup vs baseline: 4586.6814x; 3.2516x over previous
import functools

import jax
import jax.numpy as jnp
from jax import lax
from jax.experimental import pallas as pl
from jax.experimental.pallas import tpu as pltpu
from jax.experimental.pallas import tpu_sc as plsc

_EPS = 1e-6
_NB = 8
_NC = _NB + 2
_L = 16


def _make_sc_hist(B, C, nworkers):
    rows_w = B // nworkers
    CHUNK = 16
    nchunks = rows_w // CHUNK
    nsteps = nchunks // 2
    ngroups = C // _L
    acc_len = _NC * C

    mesh = plsc.VectorSubcoreMesh(core_axis_name="c", subcore_axis_name="s")

    def body(x_hbm, sb_hbm, out_hbm, xb0, xb1, sb_v, acc_v, sem0, sem1):
        ncores = 2
        wid = lax.axis_index("s") * ncores + lax.axis_index("c")
        row0 = wid * rows_w

        def chunk_copy(i, buf, sem):
            off = (row0 + i * CHUNK) * C
            return pltpu.make_async_copy(
                x_hbm.at[pl.ds(off, CHUNK * C)], buf, sem)

        chunk_copy(0, xb0, sem0).start()
        chunk_copy(1, xb1, sem1).start()
        pltpu.sync_copy(sb_hbm, sb_v)

        zeros = jnp.zeros((_L,), jnp.float32)

        @plsc.parallel_loop(0, acc_len // _L)
        def zero_body(i):
            acc_v[pl.ds(i * _L, _L)] = zeros

        lane = lax.iota(jnp.int32, _L)
        ones = jnp.full((_L,), 1.0, jnp.float32)

        def compute(buf):
            @plsc.parallel_loop(0, ngroups)
            def g_body(g):
                col0 = g * _L
                sv = sb_v[pl.ds(col0, _L)]
                bv = sb_v[pl.ds(C + col0, _L)]
                colv = col0 + lane
                fis = []
                for r in range(CHUNK):
                    v = buf[pl.ds(r * C + col0, _L)]
                    u = v * sv + bv
                    u = jnp.minimum(jnp.maximum(u, 0.0), 9.0)
                    fis.append(u.astype(jnp.int32) * C + colv)
                for fi in fis:
                    plsc.addupdate_scatter(acc_v, [fi], ones)

        def step(k, _):
            c0 = 2 * k
            chunk_copy(c0, xb0, sem0).wait()
            compute(xb0)

            @pl.when(k < nsteps - 1)
            def _start0():
                chunk_copy(c0 + 2, xb0, sem0).start()

            chunk_copy(c0 + 1, xb1, sem1).wait()
            compute(xb1)

            @pl.when(k < nsteps - 1)
            def _start1():
                chunk_copy(c0 + 3, xb1, sem1).start()

            return _

        lax.fori_loop(0, nsteps, step, None)

        pltpu.sync_copy(acc_v, out_hbm.at[wid])

    return pl.kernel(
        body,
        out_type=jax.ShapeDtypeStruct((nworkers, acc_len), jnp.float32),
        mesh=mesh,
        compiler_params=pltpu.CompilerParams(needs_layout_passes=False),
        scratch_types=[
            pltpu.VMEM((CHUNK * C,), jnp.float32),
            pltpu.VMEM((CHUNK * C,), jnp.float32),
            pltpu.VMEM((2 * C,), jnp.float32),
            pltpu.VMEM((acc_len,), jnp.float32),
            pltpu.SemaphoreType.DMA,
            pltpu.SemaphoreType.DMA,
        ],
    )


def kernel(x, mins, maxs, bin_counts, bin_edges, feature_ranges):
    B, C = x.shape
    info = plsc.get_sparse_core_info()
    nworkers = info.num_cores * info.num_subcores

    all_zero = jnp.all(bin_counts == 0.0)

    maxs2 = maxs + _EPS
    fr = maxs2 - mins
    mxs = maxs2 / fr
    mns = mins / fr
    span = mxs - mns
    scale_a = 8.0 / (fr * span)
    bias_a = 1.0 - 8.0 * mns / span

    inner = bin_edges[:, 1:-1]
    span_e = inner[:, -1] - inner[:, 0]
    safe = jnp.where(span_e != 0.0, span_e, 1.0)
    frq = feature_ranges[:, 0]
    scale_b = 8.0 / jnp.where(frq != 0.0, frq, 1.0) / safe
    bias_b = 1.0 - 8.0 * inner[:, 0] / safe

    scale = jnp.where(all_zero, scale_a, scale_b)
    bias = jnp.where(all_zero, bias_a, bias_b)
    sb = jnp.concatenate([scale, bias]).astype(jnp.float32)

    hist = _make_sc_hist(B, C, nworkers)
    parts = hist(x.reshape(-1), sb)
    counts = parts.sum(axis=0).reshape(_NC, C).T
    return (x, bin_counts + counts)


# speedup vs baseline: 5518.6284x; 1.2032x over previous
import functools

import jax
import jax.numpy as jnp
from jax import lax
from jax.experimental import pallas as pl
from jax.experimental.pallas import tpu as pltpu
from jax.experimental.pallas import tpu_sc as plsc

_EPS = 1e-6
_NB = 8
_NC = _NB + 2
_L = 16


def _make_sc_hist(B, C, nworkers):
    rows_w = B // nworkers
    CHUNK = 16
    nchunks = rows_w // CHUNK
    nsteps = nchunks // 2
    ngroups = C // _L
    acc_len = _NC * C

    mesh = plsc.VectorSubcoreMesh(core_axis_name="c", subcore_axis_name="s")

    def body(x_hbm, sb_hbm, out_hbm, xb0, xb1, sb_v, acc_v, sem0, sem1):
        ncores = 2
        wid = lax.axis_index("s") * ncores + lax.axis_index("c")
        row0 = wid * rows_w

        def chunk_copy(i, buf, sem):
            off = (row0 + i * CHUNK) * C
            return pltpu.make_async_copy(
                x_hbm.at[pl.ds(off, CHUNK * C)], buf, sem)

        chunk_copy(0, xb0, sem0).start()
        chunk_copy(1, xb1, sem1).start()
        pltpu.sync_copy(sb_hbm, sb_v)

        zeros = jnp.zeros((_L,), jnp.float32)

        @plsc.parallel_loop(0, acc_len // _L)
        def zero_body(i):
            acc_v[pl.ds(i * _L, _L)] = zeros

        lane = lax.iota(jnp.int32, _L)
        ones = jnp.full((_L,), 1.0, jnp.float32)

        def compute(buf):
            @plsc.parallel_loop(0, ngroups)
            def g_body(g):
                col0 = g * _L
                sv = sb_v[pl.ds(col0, _L)]
                bv = sb_v[pl.ds(C + col0, _L)]
                colv10 = (col0 + lane) * _NC
                fis = []
                for r in range(CHUNK):
                    v = buf[pl.ds(r * C + col0, _L)]
                    u = v * sv + bv
                    u = jnp.minimum(jnp.maximum(u, 0.0), 9.0)
                    fis.append(colv10 + u.astype(jnp.int32))
                for fi in fis:
                    plsc.addupdate_scatter(acc_v, [fi], ones)

        def step(k, _):
            c0 = 2 * k
            chunk_copy(c0, xb0, sem0).wait()
            compute(xb0)

            @pl.when(k < nsteps - 1)
            def _start0():
                chunk_copy(c0 + 2, xb0, sem0).start()

            chunk_copy(c0 + 1, xb1, sem1).wait()
            compute(xb1)

            @pl.when(k < nsteps - 1)
            def _start1():
                chunk_copy(c0 + 3, xb1, sem1).start()

            return _

        lax.fori_loop(0, nsteps, step, None)

        pltpu.sync_copy(acc_v, out_hbm.at[wid])

    return pl.kernel(
        body,
        out_type=jax.ShapeDtypeStruct((nworkers, acc_len), jnp.float32),
        mesh=mesh,
        compiler_params=pltpu.CompilerParams(needs_layout_passes=False),
        scratch_types=[
            pltpu.VMEM((CHUNK * C,), jnp.float32),
            pltpu.VMEM((CHUNK * C,), jnp.float32),
            pltpu.VMEM((2 * C,), jnp.float32),
            pltpu.VMEM((acc_len,), jnp.float32),
            pltpu.SemaphoreType.DMA,
            pltpu.SemaphoreType.DMA,
        ],
    )


def _tc_copy(x):
    B, C = x.shape
    blk = 256

    def body(x_ref, o_ref):
        o_ref[...] = x_ref[...]

    return pl.pallas_call(
        body,
        grid=(B // blk,),
        in_specs=[pl.BlockSpec((blk, C), lambda i: (i, 0))],
        out_specs=pl.BlockSpec((blk, C), lambda i: (i, 0)),
        out_shape=jax.ShapeDtypeStruct((B, C), x.dtype),
    )(x)


def kernel(x, mins, maxs, bin_counts, bin_edges, feature_ranges):
    B, C = x.shape
    info = plsc.get_sparse_core_info()
    nworkers = info.num_cores * info.num_subcores

    all_zero = jnp.all(bin_counts == 0.0)

    maxs2 = maxs + _EPS
    fr = maxs2 - mins
    mxs = maxs2 / fr
    mns = mins / fr
    span = mxs - mns
    scale_a = 8.0 / (fr * span)
    bias_a = 1.0 - 8.0 * mns / span

    inner = bin_edges[:, 1:-1]
    span_e = inner[:, -1] - inner[:, 0]
    safe = jnp.where(span_e != 0.0, span_e, 1.0)
    frq = feature_ranges[:, 0]
    scale_b = 8.0 / jnp.where(frq != 0.0, frq, 1.0) / safe
    bias_b = 1.0 - 8.0 * inner[:, 0] / safe

    scale = jnp.where(all_zero, scale_a, scale_b)
    bias = jnp.where(all_zero, bias_a, bias_b)
    sb = jnp.concatenate([scale, bias]).astype(jnp.float32)

    hist = _make_sc_hist(B, C, nworkers)
    parts = hist(x.reshape(-1), sb)
    counts = parts.sum(axis=0).reshape(C, _NC)
    x_out = _tc_copy(x)
    return (x_out, bin_counts + counts)
